# baseline (speedup 1.0000x reference)
import functools

import jax
import jax.numpy as jnp
from jax.experimental import pallas as pl
from jax.experimental.pallas import tpu as pltpu

_T = 4096
_LOGT = 12
_RBLK = 40


def _sort_score_kernel(ss_ref, sm_ref, cs_ref, act_ref, bkg_ref, ref_ref,
                       key_s, aux_s, cas_s):
    R, T = ss_ref.shape
    ss = ss_ref[...]
    sm = sm_ref[...]
    cs = cs_ref[...]

    it = jax.lax.broadcasted_iota(jnp.int32, (R, T), 1)

    bits = jax.lax.bitcast_convert_type(ss, jnp.int32)
    key_s[...] = jnp.where(bits < 0, bits ^ jnp.int32(0x7FFFFFFF), bits)
    aux_s[...] = it * 2 + sm.astype(jnp.int32)
    cas_s[...] = cs

    def layer(j, s):
        d = jnp.int32(1) << (s - j)
        s1 = s + 1
        key = key_s[...]
        aux = aux_s[...]
        csd = cas_s[...]
        am_a = (it & d) == 0
        up = ((it >> s1) & 1) == 0
        dn = T - d
        ko = jnp.where(am_a, pltpu.roll(key, dn, 1), pltpu.roll(key, d, 1))
        ao = jnp.where(am_a, pltpu.roll(aux, dn, 1), pltpu.roll(aux, d, 1))
        co = jnp.where(am_a, pltpu.roll(csd, dn, 1), pltpu.roll(csd, d, 1))
        B = (ko > key) | ((ko == key) & (ao < aux))
        take = B == (am_a == up)
        key_s[...] = jnp.where(take, ko, key)
        aux_s[...] = jnp.where(take, ao, aux)
        cas_s[...] = jnp.where(take, co, csd)
        return s

    def stage(s, _):
        jax.lax.fori_loop(0, s + 1, layer, s)
        return 0

    jax.lax.fori_loop(0, _LOGT, stage, 0)

    key_sorted = key_s[...]
    m_sorted = (aux_s[...] & 1).astype(jnp.float32)
    cas_sorted = cas_s[...]

    w = 1.0 / (it.astype(jnp.float32) + 2.0)
    act_num = (w * m_sorted).sum(axis=1, keepdims=True)
    act_raw = (cas_sorted * w * m_sorted).sum(axis=1, keepdims=True) / \
        jnp.maximum(act_num, 1.0)
    bkg_num = (1.0 - sm).sum(axis=1, keepdims=True)
    bkg_raw = (cs * (1.0 - sm)).sum(axis=1, keepdims=True) / \
        jnp.maximum(bkg_num, 1.0)

    count = (cs >= act_raw).astype(jnp.int32).sum(axis=1, keepdims=True)
    t_m = jnp.clip(count - 1, 0, T - 1)
    mean_key = jnp.where(it == t_m, key_sorted, 0).sum(axis=1, keepdims=True)
    mbits = jnp.where(mean_key < 0, mean_key ^ jnp.int32(0x7FFFFFFF), mean_key)
    mean_score = jax.lax.bitcast_convert_type(mbits, jnp.float32)

    ref_ref[...] = sm * (ss >= mean_score).astype(jnp.float32)
    act_ref[...] = act_raw
    bkg_ref[...] = bkg_raw


def _softmax_kernel(a_ref, b_ref, ao_ref, bo_ref):
    for src, dst in ((a_ref, ao_ref), (b_ref, bo_ref)):
        x = src[...]
        z = jnp.exp(x - jnp.max(x, axis=1, keepdims=True))
        dst[...] = z / jnp.sum(z, axis=1, keepdims=True)


@jax.jit
def kernel(seg_score, seg_mask, cas):
    n, t, c = seg_score.shape
    rows = n * c
    ss = jnp.transpose(seg_score, (0, 2, 1)).reshape(rows, t)
    sm = jnp.transpose(seg_mask, (0, 2, 1)).reshape(rows, t)
    cs = jnp.transpose(cas, (0, 2, 1)).reshape(rows, t)

    grid = rows // _RBLK
    row_spec = pl.BlockSpec((_RBLK, t), lambda i: (i, 0))
    col_spec = pl.BlockSpec((_RBLK, 1), lambda i: (i, 0))
    act_raw, bkg_raw, refined = pl.pallas_call(
        _sort_score_kernel,
        grid=(grid,),
        in_specs=[row_spec, row_spec, row_spec],
        out_specs=[col_spec, col_spec, row_spec],
        out_shape=[
            jax.ShapeDtypeStruct((rows, 1), jnp.float32),
            jax.ShapeDtypeStruct((rows, 1), jnp.float32),
            jax.ShapeDtypeStruct((rows, t), jnp.float32),
        ],
        scratch_shapes=[
            pltpu.VMEM((_RBLK, t), jnp.int32),
            pltpu.VMEM((_RBLK, t), jnp.int32),
            pltpu.VMEM((_RBLK, t), jnp.float32),
        ],
    )(ss, sm, cs)

    act_score, bkg_score = pl.pallas_call(
        _softmax_kernel,
        out_shape=[
            jax.ShapeDtypeStruct((n, c), jnp.float32),
            jax.ShapeDtypeStruct((n, c), jnp.float32),
        ],
    )(act_raw.reshape(n, c), bkg_raw.reshape(n, c))

    refined = refined.reshape(n, c, t).transpose(0, 2, 1)
    return act_score, bkg_score, refined
